# Initial kernel scaffold; baseline (speedup 1.0000x reference)
import jax
import jax.numpy as jnp
from jax.experimental import pallas as pl


def kernel(x, edge_index, W, b, gamma, beta):
    raise NotImplementedError("write your pallas kernel here")


# speedup vs baseline: 4.4763x; 4.4763x over previous
import dataclasses
import functools

import jax
import jax.numpy as jnp
from jax import lax
from jax.experimental import pallas as pl
from jax.experimental.pallas import tpu as pltpu
from jax.experimental.pallas import tpu_sc as plsc

NC = 2
NS = 16
NW = NC * NS
WIN = 128
LANES = 16
IDX_CHUNK = 16


def _round_up(a, m):
    return (a + m - 1) // m * m


@functools.partial(jax.jit, static_argnames=("n", "np_", "steps", "d"))
def _sc_accumulate(x, src_t, dst_t, *, n, np_, steps, d):
    rows_per_tile = np_ // NS
    zchunks = rows_per_tile // WIN
    zrem = rows_per_tile % WIN
    ichunks = steps // IDX_CHUNK

    mesh = plsc.VectorSubcoreMesh(core_axis_name="c", subcore_axis_name="s")
    cp = pltpu.CompilerParams()
    if "needs_layout_passes" in pltpu.CompilerParams.__dataclass_fields__:
        cp = dataclasses.replace(cp, needs_layout_passes=False)

    @functools.partial(
        pl.kernel,
        mesh=mesh,
        compiler_params=cp,
        out_type=[
            jax.ShapeDtypeStruct((NC, np_, d), jnp.float32),
            jax.ShapeDtypeStruct((NW, np_), jnp.float32),
        ],
        scratch_types=[
            pltpu.VMEM_SHARED((np_, d), jnp.float32),
            pltpu.VMEM((WIN, d), jnp.float32),
            pltpu.VMEM((np_,), jnp.float32),
            pltpu.VMEM((IDX_CHUNK, WIN), jnp.int32),
            pltpu.VMEM((IDX_CHUNK, WIN), jnp.int32),
            pltpu.SemaphoreType.DMA,
        ],
    )
    def sc_kernel(x_hbm, src_hbm, dst_hbm, ah_out, deg_out,
                  ah_sh, rows_v, deg_v, sidx, didx, sem):
        c = lax.axis_index("c")
        s = lax.axis_index("s")
        wid = c * NS + s
        row0 = s * rows_per_tile

        zeros16 = jnp.zeros((LANES,), jnp.float32)
        ones16 = jnp.ones((LANES,), jnp.float32)

        @pl.loop(0, WIN)
        def _(i):
            @pl.loop(0, d // LANES)
            def _(j):
                rows_v[i, pl.ds(j * LANES, LANES)] = zeros16

        @pl.loop(0, np_ // LANES)
        def _(i):
            deg_v[pl.ds(i * LANES, LANES)] = zeros16

        @pl.loop(0, zchunks)
        def _(i):
            pltpu.sync_copy(rows_v, ah_sh.at[pl.ds(row0 + i * WIN, WIN)])
        if zrem:
            pltpu.sync_copy(rows_v.at[pl.ds(0, zrem)],
                            ah_sh.at[pl.ds(row0 + zchunks * WIN, zrem)])

        plsc.subcore_barrier()

        @pl.loop(0, ichunks)
        def _(cc):
            pltpu.sync_copy(src_hbm.at[wid, pl.ds(cc * IDX_CHUNK, IDX_CHUNK)],
                            sidx)
            pltpu.sync_copy(dst_hbm.at[wid, pl.ds(cc * IDX_CHUNK, IDX_CHUNK)],
                            didx)

            @pl.loop(0, IDX_CHUNK)
            def _(t):
                pltpu.async_copy(x_hbm.at[sidx.at[t]], rows_v, sem).wait()
                pltpu.sync_copy(rows_v, ah_sh.at[didx.at[t]], add=True)

                @pl.loop(0, WIN // LANES)
                def _(k):
                    idx = didx[t, pl.ds(k * LANES, LANES)]
                    plsc.addupdate_scatter(deg_v, [idx], ones16)

        plsc.subcore_barrier()

        @pl.loop(0, zchunks)
        def _(i):
            base = row0 + i * WIN
            pltpu.sync_copy(ah_sh.at[pl.ds(base, WIN)],
                            ah_out.at[c, pl.ds(base, WIN)])
        if zrem:
            base = row0 + zchunks * WIN
            pltpu.sync_copy(ah_sh.at[pl.ds(base, zrem)],
                            ah_out.at[c, pl.ds(base, zrem)])
        pltpu.sync_copy(deg_v, deg_out.at[wid])

    return sc_kernel(x, src_t, dst_t)


def _tc_body(ah_ref, deg_ref, w_ref, b_ref, g_ref, be_ref, o_ref):
    ah = ah_ref[0] + ah_ref[1]
    deg = jnp.sum(deg_ref[...], axis=1, keepdims=True)
    norm = jnp.where(deg > 0.0, 1.0 / deg, 0.0)
    h = jnp.dot(ah * norm, w_ref[...], preferred_element_type=jnp.float32)
    h = h + b_ref[...]
    mean = jnp.mean(h, axis=1, keepdims=True)
    zc = h - mean
    var = jnp.mean(zc * zc, axis=1, keepdims=True)
    h = zc / jnp.sqrt(var + 1e-5) * g_ref[...] + be_ref[...]
    o_ref[...] = jnp.maximum(h, 0.0)


def kernel(x, edge_index, W, b, gamma, beta):
    n, din = x.shape
    dout = W.shape[1]
    e = edge_index.shape[1]

    steps = _round_up(-(-e // (NW * WIN)), IDX_CHUNK)
    ep = steps * NW * WIN
    np_ = _round_up(n + 1, NS * 8)

    src = edge_index[0].astype(jnp.int32)
    dst = edge_index[1].astype(jnp.int32)
    pad = ep - e
    src_p = jnp.concatenate([src, jnp.zeros((pad,), jnp.int32)])
    dst_p = jnp.concatenate([dst, jnp.full((pad,), n, jnp.int32)])
    src_t = src_p.reshape(steps, NW, WIN).transpose(1, 0, 2)
    dst_t = dst_p.reshape(steps, NW, WIN).transpose(1, 0, 2)

    ah_p, deg_p = _sc_accumulate(x, src_t, dst_t,
                                 n=n, np_=np_, steps=steps, d=din)
    deg_p = deg_p.T

    rblk = 1000 if n % 1000 == 0 else n
    grid = n // rblk
    out = pl.pallas_call(
        _tc_body,
        grid=(grid,),
        in_specs=[
            pl.BlockSpec((NC, rblk, din), lambda i: (0, i, 0)),
            pl.BlockSpec((rblk, NW), lambda i: (i, 0)),
            pl.BlockSpec((din, dout), lambda i: (0, 0)),
            pl.BlockSpec((1, dout), lambda i: (0, 0)),
            pl.BlockSpec((1, dout), lambda i: (0, 0)),
            pl.BlockSpec((1, dout), lambda i: (0, 0)),
        ],
        out_specs=pl.BlockSpec((rblk, dout), lambda i: (i, 0)),
        out_shape=jax.ShapeDtypeStruct((n, dout), jnp.float32),
    )(ah_p, deg_p, W, b.reshape(1, dout), gamma.reshape(1, dout),
      beta.reshape(1, dout))
    return out


# speedup vs baseline: 4.8958x; 1.0937x over previous
import dataclasses
import functools

import jax
import jax.numpy as jnp
from jax import lax
from jax.experimental import pallas as pl
from jax.experimental.pallas import tpu as pltpu
from jax.experimental.pallas import tpu_sc as plsc

NC = 2
NS = 16
NW = NC * NS
WIN = 128
LANES = 16
IDX_CHUNK = 16


def _round_up(a, m):
    return (a + m - 1) // m * m


@functools.partial(jax.jit, static_argnames=("n", "np_", "steps", "d"))
def _sc_accumulate(x, src_t, dst_t, *, n, np_, steps, d):
    rows_per_tile = np_ // NS
    zchunks = rows_per_tile // WIN
    zrem = rows_per_tile % WIN
    ichunks = steps // IDX_CHUNK

    mesh = plsc.VectorSubcoreMesh(core_axis_name="c", subcore_axis_name="s")
    cp = pltpu.CompilerParams()
    if "needs_layout_passes" in pltpu.CompilerParams.__dataclass_fields__:
        cp = dataclasses.replace(cp, needs_layout_passes=False)

    @functools.partial(
        pl.kernel,
        mesh=mesh,
        compiler_params=cp,
        out_type=[
            jax.ShapeDtypeStruct((NC, np_, d), jnp.float32),
            jax.ShapeDtypeStruct((NW, np_), jnp.float32),
        ],
        scratch_types=[
            pltpu.VMEM_SHARED((np_, d), jnp.float32),
            pltpu.VMEM((WIN, d), jnp.float32),
            pltpu.VMEM((WIN, d), jnp.float32),
            pltpu.VMEM((np_,), jnp.float32),
            pltpu.VMEM((IDX_CHUNK, WIN), jnp.int32),
            pltpu.VMEM((IDX_CHUNK, WIN), jnp.int32),
            pltpu.SemaphoreType.DMA,
            pltpu.SemaphoreType.DMA,
        ],
    )
    def sc_kernel(x_hbm, src_hbm, dst_hbm, ah_out, deg_out,
                  ah_sh, rows_v, rows_w, deg_v, sidx, didx, sem0, sem1):
        c = lax.axis_index("c")
        s = lax.axis_index("s")
        wid = c * NS + s
        row0 = s * rows_per_tile

        zeros16 = jnp.zeros((LANES,), jnp.float32)
        ones16 = jnp.ones((LANES,), jnp.float32)

        @pl.loop(0, WIN)
        def _(i):
            @pl.loop(0, d // LANES)
            def _(j):
                rows_v[i, pl.ds(j * LANES, LANES)] = zeros16

        @pl.loop(0, np_ // LANES)
        def _(i):
            deg_v[pl.ds(i * LANES, LANES)] = zeros16

        @pl.loop(0, zchunks)
        def _(i):
            pltpu.sync_copy(rows_v, ah_sh.at[pl.ds(row0 + i * WIN, WIN)])
        if zrem:
            pltpu.sync_copy(rows_v.at[pl.ds(0, zrem)],
                            ah_sh.at[pl.ds(row0 + zchunks * WIN, zrem)])

        plsc.subcore_barrier()

        slots = (rows_v, rows_w)
        sems = (sem0, sem1)

        @pl.loop(0, ichunks)
        def _(cc):
            pltpu.sync_copy(src_hbm.at[wid, pl.ds(cc * IDX_CHUNK, IDX_CHUNK)],
                            sidx)
            pltpu.sync_copy(dst_hbm.at[wid, pl.ds(cc * IDX_CHUNK, IDX_CHUNK)],
                            didx)

            handle = pltpu.async_copy(x_hbm.at[sidx.at[0]], slots[0], sems[0])
            for t in range(IDX_CHUNK):
                cur = slots[t % 2]
                handle.wait()
                if t + 1 < IDX_CHUNK:
                    handle = pltpu.async_copy(x_hbm.at[sidx.at[t + 1]],
                                              slots[(t + 1) % 2],
                                              sems[(t + 1) % 2])
                for k in range(WIN // LANES):
                    idx = didx[t, pl.ds(k * LANES, LANES)]
                    plsc.addupdate_scatter(deg_v, [idx], ones16)
                pltpu.sync_copy(cur, ah_sh.at[didx.at[t]], add=True)

        plsc.subcore_barrier()

        @pl.loop(0, zchunks)
        def _(i):
            base = row0 + i * WIN
            pltpu.sync_copy(ah_sh.at[pl.ds(base, WIN)],
                            ah_out.at[c, pl.ds(base, WIN)])
        if zrem:
            base = row0 + zchunks * WIN
            pltpu.sync_copy(ah_sh.at[pl.ds(base, zrem)],
                            ah_out.at[c, pl.ds(base, zrem)])
        pltpu.sync_copy(deg_v, deg_out.at[wid])

    return sc_kernel(x, src_t, dst_t)


def _tc_body(ah_ref, deg_ref, w_ref, b_ref, g_ref, be_ref, o_ref):
    ah = ah_ref[0] + ah_ref[1]
    deg = jnp.sum(deg_ref[...], axis=1, keepdims=True)
    norm = jnp.where(deg > 0.0, 1.0 / deg, 0.0)
    h = jnp.dot(ah * norm, w_ref[...], preferred_element_type=jnp.float32)
    h = h + b_ref[...]
    mean = jnp.mean(h, axis=1, keepdims=True)
    zc = h - mean
    var = jnp.mean(zc * zc, axis=1, keepdims=True)
    h = zc / jnp.sqrt(var + 1e-5) * g_ref[...] + be_ref[...]
    o_ref[...] = jnp.maximum(h, 0.0)


def kernel(x, edge_index, W, b, gamma, beta):
    n, din = x.shape
    dout = W.shape[1]
    e = edge_index.shape[1]

    steps = _round_up(-(-e // (NW * WIN)), IDX_CHUNK)
    ep = steps * NW * WIN
    np_ = _round_up(n + 1, NS * 8)

    src = edge_index[0].astype(jnp.int32)
    dst = edge_index[1].astype(jnp.int32)
    pad = ep - e
    src_p = jnp.concatenate([src, jnp.zeros((pad,), jnp.int32)])
    dst_p = jnp.concatenate([dst, jnp.full((pad,), n, jnp.int32)])
    src_t = src_p.reshape(steps, NW, WIN).transpose(1, 0, 2)
    dst_t = dst_p.reshape(steps, NW, WIN).transpose(1, 0, 2)

    ah_p, deg_p = _sc_accumulate(x, src_t, dst_t,
                                 n=n, np_=np_, steps=steps, d=din)
    deg_p = deg_p.T

    rblk = 1000 if n % 1000 == 0 else n
    grid = n // rblk
    out = pl.pallas_call(
        _tc_body,
        grid=(grid,),
        in_specs=[
            pl.BlockSpec((NC, rblk, din), lambda i: (0, i, 0)),
            pl.BlockSpec((rblk, NW), lambda i: (i, 0)),
            pl.BlockSpec((din, dout), lambda i: (0, 0)),
            pl.BlockSpec((1, dout), lambda i: (0, 0)),
            pl.BlockSpec((1, dout), lambda i: (0, 0)),
            pl.BlockSpec((1, dout), lambda i: (0, 0)),
        ],
        out_specs=pl.BlockSpec((rblk, dout), lambda i: (i, 0)),
        out_shape=jax.ShapeDtypeStruct((n, dout), jnp.float32),
    )(ah_p, deg_p, W, b.reshape(1, dout), gamma.reshape(1, dout),
      beta.reshape(1, dout))
    return out
